# baseline (speedup 1.0000x reference)
import functools

import jax
import jax.numpy as jnp
from jax import lax
from jax.experimental import pallas as pl
from jax.experimental.pallas import tpu as pltpu

B, N, HID = 16, 2048, 256
ROWS = B * N
BLK = 2048
NBLK = ROWS // BLK
NEG = -1e30


def _phase1_body(emb_ref, plen_ref, w1t_ref, b1_ref, w2_ref,
                 out_ref, pooled_ref, m_ref, z_ref, v_ref):
    i = pl.program_id(0)

    @pl.when(i == 0)
    def _init():
        m_ref[0, 0] = NEG
        z_ref[0, 0] = 0.0
        v_ref[...] = jnp.zeros_like(v_ref)

    e = emb_ref[...]
    plen = plen_ref[...]
    h = jnp.tanh(jnp.dot(e, w1t_ref[...],
                         preferred_element_type=jnp.float32) + b1_ref[...])
    s = jnp.sum(h * w2_ref[...], axis=1, keepdims=True)
    med = (plen >= 3) & (plen < 5)
    short = plen < 3
    sm = jnp.where(med, s, NEG)
    bm = jnp.max(sm)
    m_old = m_ref[0, 0]
    m_new = jnp.maximum(m_old, bm)
    alpha = jnp.exp(m_old - m_new)
    p = jnp.where(med, jnp.exp(sm - m_new), 0.0)
    z_ref[0, 0] = z_ref[0, 0] * alpha + jnp.sum(p)
    pv = lax.dot_general(p, e, (((0,), (0,)), ((), ())),
                         preferred_element_type=jnp.float32)
    v_ref[...] = v_ref[...] * alpha + pv
    m_ref[0, 0] = m_new
    out_ref[...] = jnp.where(short, e, 0.0)

    @pl.when(i == NBLK - 1)
    def _fin():
        pooled_ref[...] = v_ref[...] / z_ref[0, 0]


def _phase2_body(out1_ref, plen_ref, pooled_ref, out_ref):
    plen = plen_ref[...]
    med = (plen >= 3) & (plen < 5)
    out_ref[...] = jnp.where(med, pooled_ref[...], out1_ref[...])


def kernel(embeddings, path_lengths, W1, b1, W2, b2):
    del b2
    emb2 = embeddings.reshape(ROWS, HID)
    plen2 = path_lengths.reshape(ROWS, 1)
    w1t = W1.T
    b1r = b1.reshape(1, HID // 2)
    w2r = W2.reshape(1, HID // 2)

    out1, pooled = pl.pallas_call(
        _phase1_body,
        grid=(NBLK,),
        in_specs=[
            pl.BlockSpec((BLK, HID), lambda i: (i, 0)),
            pl.BlockSpec((BLK, 1), lambda i: (i, 0)),
            pl.BlockSpec((HID, HID // 2), lambda i: (0, 0)),
            pl.BlockSpec((1, HID // 2), lambda i: (0, 0)),
            pl.BlockSpec((1, HID // 2), lambda i: (0, 0)),
        ],
        out_specs=[
            pl.BlockSpec((BLK, HID), lambda i: (i, 0)),
            pl.BlockSpec((1, HID), lambda i: (0, 0)),
        ],
        out_shape=[
            jax.ShapeDtypeStruct((ROWS, HID), jnp.float32),
            jax.ShapeDtypeStruct((1, HID), jnp.float32),
        ],
        scratch_shapes=[
            pltpu.SMEM((1, 1), jnp.float32),
            pltpu.SMEM((1, 1), jnp.float32),
            pltpu.VMEM((1, HID), jnp.float32),
        ],
    )(emb2, plen2, w1t, b1r, w2r)

    out = pl.pallas_call(
        _phase2_body,
        grid=(NBLK,),
        in_specs=[
            pl.BlockSpec((BLK, HID), lambda i: (i, 0)),
            pl.BlockSpec((BLK, 1), lambda i: (i, 0)),
            pl.BlockSpec((1, HID), lambda i: (0, 0)),
        ],
        out_specs=pl.BlockSpec((BLK, HID), lambda i: (i, 0)),
        out_shape=jax.ShapeDtypeStruct((ROWS, HID), jnp.float32),
        input_output_aliases={0: 0},
    )(out1, plen2, pooled)

    return out.reshape(B, N, HID)


# speedup vs baseline: 1.2273x; 1.2273x over previous
import functools

import jax
import jax.numpy as jnp
from jax import lax
from jax.experimental import pallas as pl
from jax.experimental.pallas import tpu as pltpu

B, N, HID = 16, 2048, 256
ROWS = B * N
BLK = 2048
NBLK = ROWS // BLK
NEG = -1e30


def _body(emb_ref, plen_ref, w1t_ref, b1_ref, w2_ref,
          out_ref, m_ref, z_ref, v_ref, esave_ref, pooled_ref):
    j = pl.program_id(0)
    i = pl.program_id(1)

    @pl.when((j == 0) & (i == 0))
    def _init():
        m_ref[0, 0] = NEG
        z_ref[0, 0] = 0.0
        v_ref[...] = jnp.zeros_like(v_ref)

    @pl.when(j == 0)
    def _accumulate():
        e = emb_ref[...]
        plen = plen_ref[...]
        esave_ref[pl.ds(i * BLK, BLK), :] = e
        h = jnp.tanh(jnp.dot(e, w1t_ref[...],
                             preferred_element_type=jnp.float32) + b1_ref[...])
        s = jnp.sum(h * w2_ref[...], axis=1, keepdims=True)
        med = (plen >= 3) & (plen < 5)
        sm = jnp.where(med, s, NEG)
        bm = jnp.max(sm)
        m_old = m_ref[0, 0]
        m_new = jnp.maximum(m_old, bm)
        alpha = jnp.exp(m_old - m_new)
        p = jnp.where(med, jnp.exp(sm - m_new), 0.0)
        z_ref[0, 0] = z_ref[0, 0] * alpha + jnp.sum(p)
        pv = lax.dot_general(p, e, (((0,), (0,)), ((), ())),
                             preferred_element_type=jnp.float32)
        v_ref[...] = v_ref[...] * alpha + pv
        m_ref[0, 0] = m_new

        @pl.when(i == NBLK - 1)
        def _fin():
            pooled_ref[...] = v_ref[...] / z_ref[0, 0]

    @pl.when(j == 1)
    def _emit():
        e = esave_ref[pl.ds(i * BLK, BLK), :]
        plen = plen_ref[...]
        med = (plen >= 3) & (plen < 5)
        short = plen < 3
        out_ref[...] = jnp.where(short, e,
                                 jnp.where(med, pooled_ref[...], 0.0))


def kernel(embeddings, path_lengths, W1, b1, W2, b2):
    del b2
    emb2 = embeddings.reshape(ROWS, HID)
    plen2 = path_lengths.reshape(ROWS, 1)
    w1t = W1.T
    b1r = b1.reshape(1, HID // 2)
    w2r = W2.reshape(1, HID // 2)

    out = pl.pallas_call(
        _body,
        grid=(2, NBLK),
        in_specs=[
            pl.BlockSpec((BLK, HID),
                         lambda j, i: (jnp.where(j == 0, i, NBLK - 1), 0)),
            pl.BlockSpec((BLK, 1), lambda j, i: (i, 0)),
            pl.BlockSpec((HID, HID // 2), lambda j, i: (0, 0)),
            pl.BlockSpec((1, HID // 2), lambda j, i: (0, 0)),
            pl.BlockSpec((1, HID // 2), lambda j, i: (0, 0)),
        ],
        out_specs=pl.BlockSpec((BLK, HID),
                               lambda j, i: (jnp.where(j == 0, 0, i), 0)),
        out_shape=jax.ShapeDtypeStruct((ROWS, HID), jnp.float32),
        scratch_shapes=[
            pltpu.SMEM((1, 1), jnp.float32),
            pltpu.SMEM((1, 1), jnp.float32),
            pltpu.VMEM((1, HID), jnp.float32),
            pltpu.VMEM((ROWS, HID), jnp.float32),
            pltpu.VMEM((1, HID), jnp.float32),
        ],
        compiler_params=pltpu.CompilerParams(
            dimension_semantics=("arbitrary", "arbitrary"),
        ),
    )(emb2, plen2, w1t, b1r, w2r)

    return out.reshape(B, N, HID)
